# Initial kernel scaffold; baseline (speedup 1.0000x reference)
import jax
import jax.numpy as jnp
from jax.experimental import pallas as pl


def kernel(embeddings, labels):
    raise NotImplementedError("write your pallas kernel here")


# speedup vs baseline: 1601.7880x; 1601.7880x over previous
import functools

import jax
import jax.numpy as jnp
from jax.experimental import pallas as pl

N = 4096
D = 128
TOPK = 5
BLK = 256
NB = N // BLK
BIG = 3.0e38


def _cmc_body(erow_ref, efull_ref, labr_ref, labf_ref, out_ref):
    i = pl.program_id(0)
    er = erow_ref[...]
    ef = efull_ref[...]
    labr = labr_ref[...]
    labf = labf_ref[...]

    sq_full = jnp.sum(ef * ef, axis=1)
    sq_rows = jnp.sum(er * er, axis=1)
    dot = jax.lax.dot_general(
        er, ef, (((1,), (1,)), ((), ())),
        preferred_element_type=jnp.float32)
    dist = sq_rows[:, None] + sq_full[None, :] - 2.0 * dot

    col = jax.lax.broadcasted_iota(jnp.int32, (BLK, N), 1)
    rowg = i * BLK + jax.lax.broadcasted_iota(jnp.int32, (BLK, N), 0)
    dist = jnp.where(col == rowg, BIG, dist)

    lab_match = labr[0, :][:, None] == labf[0, :][None, :]

    match = jnp.zeros((BLK, 1), dtype=jnp.bool_)
    for _ in range(TOPK):
        m = jnp.min(dist, axis=1, keepdims=True)
        is_min = dist == m
        match = match | jnp.any(is_min & lab_match, axis=1, keepdims=True)
        dist = jnp.where(is_min, BIG, dist)

    cnt = jnp.sum(match.astype(jnp.float32), axis=0, keepdims=True)
    prev = jnp.where(i == 0, jnp.zeros((1, 1), jnp.float32), out_ref[...])
    total = prev + cnt
    out_ref[...] = jnp.where(i == NB - 1, total / jnp.float32(N), total)


@jax.jit
def kernel(embeddings, labels):
    labels2 = labels.reshape(1, N)
    out = pl.pallas_call(
        _cmc_body,
        grid=(NB,),
        in_specs=[
            pl.BlockSpec((BLK, D), lambda i: (i, 0)),
            pl.BlockSpec((N, D), lambda i: (0, 0)),
            pl.BlockSpec((1, BLK), lambda i: (0, i)),
            pl.BlockSpec((1, N), lambda i: (0, 0)),
        ],
        out_specs=pl.BlockSpec((1, 1), lambda i: (0, 0)),
        out_shape=jax.ShapeDtypeStruct((1, 1), jnp.float32),
    )(embeddings, embeddings, labels2, labels2)
    return out.reshape(())
